# baseline (speedup 1.0000x reference)
import jax
import jax.numpy as jnp
from jax.experimental import pallas as pl
from jax.experimental.pallas import tpu as pltpu


def _mm_body(a0_ref, a1_ref, w_ref, out_ref):
    w = w_ref[...]
    out_ref[0] = jnp.dot(a0_ref[0], w, preferred_element_type=jnp.float32)
    out_ref[1] = jnp.dot(a1_ref[0], w, preferred_element_type=jnp.float32)


def kernel(adj, weight):
    m, k = adj.shape
    k2, n = weight.shape
    assert k == k2
    bm = 128
    half = m // 2
    adj3 = adj.reshape(2, half, k)
    grid = (half // bm,)
    out3 = pl.pallas_call(
        _mm_body,
        grid=grid,
        in_specs=[
            pl.BlockSpec((1, bm, k), lambda i: (0, i, 0)),
            pl.BlockSpec((1, bm, k), lambda i: (1, i, 0)),
            pl.BlockSpec((k2, n), lambda i: (0, 0)),
        ],
        out_specs=pl.BlockSpec((2, bm, n), lambda i: (0, i, 0)),
        out_shape=jax.ShapeDtypeStruct((2, half, n), jnp.float32),
        compiler_params=pltpu.CompilerParams(
            dimension_semantics=("arbitrary",),
        ),
    )(adj3, adj3, weight)
    return out3.reshape(m, n)
